# Initial kernel scaffold; baseline (speedup 1.0000x reference)
import jax
import jax.numpy as jnp
from jax.experimental import pallas as pl


def kernel(input, table):
    raise NotImplementedError("write your pallas kernel here")


# speedup vs baseline: 1.0771x; 1.0771x over previous
import functools

import jax
import jax.numpy as jnp
from jax import lax
from jax.experimental import pallas as pl
from jax.experimental.pallas import tpu as pltpu
from jax.experimental.pallas import tpu_sc as plsc

LATENT = 32
NC, NS = 2, 16
NW = NC * NS
B = 16384 * 50
B_PER_W = B // NW
CH = 1600
NCH = B_PER_W // CH
NBUF = 2

_mesh = plsc.VectorSubcoreMesh(core_axis_name="c", subcore_axis_name="s")


@functools.partial(
    pl.kernel,
    out_type=jax.ShapeDtypeStruct((B, LATENT), jnp.float32),
    mesh=_mesh,
    compiler_params=pltpu.CompilerParams(use_tc_tiling_on_sc=False),
    scratch_types=[
        pltpu.VMEM((CH,), jnp.int32),
        pltpu.VMEM((CH,), jnp.int32),
        pltpu.VMEM((NBUF, CH, LATENT), jnp.float32),
        pltpu.SemaphoreType.DMA,
    ],
)
def _emb_gather(idx_hbm, table_hbm, out_hbm, idx_v0, idx_v1, rows_v, gsem):
    wid = lax.axis_index("s") * NC + lax.axis_index("c")
    base = wid * B_PER_W
    idx_bufs = (idx_v0, idx_v1)

    def load_and_fire(c, b):
        pltpu.sync_copy(idx_hbm.at[pl.ds(base + c * CH, CH)], idx_bufs[b])
        pltpu.async_copy(table_hbm.at[idx_bufs[b]], rows_v.at[b], gsem)

    def drain_and_store(c, b):
        pltpu.make_async_copy(table_hbm.at[idx_bufs[b]], rows_v.at[b], gsem).wait()
        pltpu.sync_copy(rows_v.at[b], out_hbm.at[pl.ds(base + c * CH, CH)])

    for b in range(NBUF):
        load_and_fire(b, b)

    @pl.loop(0, NCH - NBUF, step=NBUF)
    def _(g):
        for b in range(NBUF):
            drain_and_store(g + b, b)
            load_and_fire(g + b + NBUF, b)

    for b in range(NBUF):
        drain_and_store(NCH - NBUF + b, b)


def kernel(input, table):
    idx = input.reshape(-1).astype(jnp.int32)
    return _emb_gather(idx, table)
